# baseline (speedup 1.0000x reference)
import functools

import jax
import jax.numpy as jnp
from jax import lax
from jax.experimental import pallas as pl
from jax.experimental.pallas import tpu as pltpu
from jax.experimental.pallas import tpu_sc as plsc

_EPS = 1e-5
_NC, _NS = 2, 16
_NW = _NC * _NS
_LANES = 16


@functools.lru_cache(maxsize=None)
def _make_pool(B, L, D):
    SPW = B // _NW
    NB = 4
    NS = 4
    NCHUNK = SPW // NB
    CROWS = NB * L
    WINDOWS = [(0, 128), (128, 72)]
    NGRP = D // 32
    mesh = plsc.VectorSubcoreMesh(core_axis_name="c", subcore_axis_name="s")

    @functools.partial(
        pl.kernel,
        mesh=mesh,
        compiler_params=pltpu.CompilerParams(
            use_tc_tiling_on_sc=False, needs_layout_passes=False),
        out_type=jax.ShapeDtypeStruct((B * D,), jnp.float32),
        scratch_types=[
            pltpu.VMEM((NS, CROWS), jnp.int32),
            pltpu.VMEM((NS, CROWS, D), jnp.bfloat16),
            pltpu.VMEM((NS, NB * D), jnp.float32),
            pltpu.SemaphoreType.DMA((NS,)),
            pltpu.SemaphoreType.DMA((NS,)),
            pltpu.SemaphoreType.DMA((NS,)),
        ],
    )
    def pool(idx_hbm, emb_hbm, out_hbm, idx_v, rows_v, out_v, isem, gsem, osem):
        wid = lax.axis_index("c") * _NS + lax.axis_index("s")
        base = wid * SPW
        scale = 1.0 / L

        def idx_copy(c, slot):
            pltpu.async_copy(
                idx_hbm.at[pl.ds((base + c * NB) * L, CROWS)],
                idx_v.at[slot], isem.at[slot])

        def idx_wait(slot):
            pltpu.make_async_copy(
                idx_hbm.at[pl.ds(0, CROWS)], idx_v.at[slot], isem.at[slot]
            ).wait()

        def issue_gathers(slot):
            for i in range(NB):
                for off, w in WINDOWS:
                    pltpu.async_copy(
                        emb_hbm.at[idx_v.at[slot, pl.ds(i * L + off, w)]],
                        rows_v.at[slot, pl.ds(i * L + off, w)],
                        gsem.at[slot])

        def drain_gathers(slot):
            pltpu.make_async_copy(
                emb_hbm.at[pl.ds(0, CROWS)], rows_v.at[slot], gsem.at[slot]
            ).wait()

        def store_wait(slot):
            pltpu.make_async_copy(
                out_v.at[slot], out_hbm.at[pl.ds(0, NB * D)], osem.at[slot]
            ).wait()

        for k in range(NS - 1):
            pltpu.sync_copy(
                idx_hbm.at[pl.ds((base + k * NB) * L, CROWS)], idx_v.at[k])
            issue_gathers(k)
        idx_copy(NS - 1, NS - 1)

        @pl.loop(0, NCHUNK, step=NS)
        def _outer(it):
            for p in range(NS):
                c = it + p
                q = (p + NS - 1) % NS

                @pl.when(c + NS - 1 < NCHUNK)
                def _fire_ahead(q=q):
                    idx_wait(q)
                    issue_gathers(q)

                drain_gathers(p)

                @pl.when(c + NS < NCHUNK)
                def _prefetch_idx(c=c, p=p):
                    idx_copy(c + NS, p)

                @pl.when(c >= NS)
                def _drain_prev_store(p=p):
                    store_wait(p)

                for i in range(NB):
                    zero = jnp.zeros((_LANES,), jnp.float32)

                    def rbody(t, accs, i=i, p=p):
                        a = list(accs)
                        r0 = i * L + t * 8
                        for u in range(8):
                            for g in range(NGRP):
                                v = rows_v[p, r0 + u, pl.ds(g * 32, 32)]
                                ev, od = plsc.unpack(
                                    v, format=plsc.PackFormat.INTERLEAVED)
                                a[2 * g] = a[2 * g] + ev
                                a[2 * g + 1] = a[2 * g + 1] + od
                        return tuple(a)

                    accs = lax.fori_loop(0, L // 8, rbody, (zero,) * (2 * NGRP))
                    for k in range(2 * NGRP):
                        out_v[p, pl.ds(i * D + k * _LANES, _LANES)] = (
                            accs[k] * scale)

                pltpu.async_copy(
                    out_v.at[p],
                    out_hbm.at[pl.ds((base + c * NB) * D, NB * D)],
                    osem.at[p])

        for slot in range(NS):
            store_wait(slot)

    return pool


def _mlp(pooled2, W1, W2, Wout, bout, alpha, g1, b1, g2, b2):
    R = pooled2.shape[0]
    D = W1.shape[1]
    F1 = W1.shape[0]
    F2 = W2.shape[0]
    NCLS = Wout.shape[0]
    NP = 32
    bn = (1.0 + _EPS) ** -0.5
    wout_p = jnp.zeros((NP, F2), jnp.float32).at[:NCLS].set(Wout)
    bout_2 = bout.reshape(1, NCLS)
    s1 = (g1 * bn).reshape(1, F1)
    s2 = (g2 * bn).reshape(1, F2)
    b1_2 = b1.reshape(1, F1)
    b2_2 = b2.reshape(1, F2)
    a_2 = alpha.reshape(1, 1)
    bm = 1024

    def half(x, al, w1_ref, w2_ref, wo_ref, s1_ref, b1_ref, s2_ref, b2_ref,
             bo_ref):
        h = lax.dot_general(x, w1_ref[...], (((1,), (1,)), ((), ())),
                            preferred_element_type=jnp.float32)
        h = jnp.where(h > 0, h, al * h)
        h = h * s1_ref[...] + b1_ref[...]
        h = lax.dot_general(h, w2_ref[...], (((1,), (1,)), ((), ())),
                            preferred_element_type=jnp.float32)
        h = jnp.where(h > 0, h, al * h)
        h = h * s2_ref[...] + b2_ref[...]
        o = lax.dot_general(h, wo_ref[...], (((1,), (1,)), ((), ())),
                            preferred_element_type=jnp.float32)
        return o[:, :NCLS] + bo_ref[...]

    def body(x_ref, w1_ref, w2_ref, wo_ref, s1_ref, b1_ref, s2_ref, b2_ref,
             bo_ref, a_ref, o_ref):
        x = x_ref[...]
        al = a_ref[0, 0]
        args = (al, w1_ref, w2_ref, wo_ref, s1_ref, b1_ref, s2_ref, b2_ref,
                bo_ref)
        oa = half(x[:, :D], *args)
        ob = half(x[:, D:], *args)
        o_ref[...] = jnp.concatenate([oa, ob], axis=1)

    out = pl.pallas_call(
        body,
        grid=(R // bm,),
        in_specs=[
            pl.BlockSpec((bm, 2 * D), lambda i: (i, 0)),
            pl.BlockSpec((F1, D), lambda i: (0, 0)),
            pl.BlockSpec((F2, F1), lambda i: (0, 0)),
            pl.BlockSpec((NP, F2), lambda i: (0, 0)),
            pl.BlockSpec((1, F1), lambda i: (0, 0)),
            pl.BlockSpec((1, F1), lambda i: (0, 0)),
            pl.BlockSpec((1, F2), lambda i: (0, 0)),
            pl.BlockSpec((1, F2), lambda i: (0, 0)),
            pl.BlockSpec((1, NCLS), lambda i: (0, 0)),
            pl.BlockSpec((1, 1), lambda i: (0, 0)),
        ],
        out_specs=pl.BlockSpec((bm, 2 * NCLS), lambda i: (i, 0)),
        out_shape=jax.ShapeDtypeStruct((R, 2 * NCLS), jnp.float32),
    )(pooled2, W1, W2, wout_p, s1, b1_2, s2, b2_2, bout_2, a_2)
    return out


def kernel(input, embed, W1, W2, Wout, bout, alpha, g1, b1, g2, b2):
    B, L = input.shape
    V, D = embed.shape
    idx_lin = input.reshape(B * L).astype(jnp.int32)
    emb_lin = embed.astype(jnp.bfloat16).reshape(V * D)
    perm = []
    for g in range(D // 32):
        perm += [g * 32 + 2 * r for r in range(16)]
        perm += [g * 32 + 2 * r + 1 for r in range(16)]
    w1p = W1[:, jnp.array(perm)]
    pooled_lin = _make_pool(B, L, D)(idx_lin, emb_lin.reshape(V, D))
    out2 = _mlp(pooled_lin.reshape(B // 2, 2 * D), w1p, W2, Wout, bout,
                alpha, g1, b1, g2, b2)
    NCLS = Wout.shape[0]
    return jnp.stack(
        [out2[:, :NCLS], out2[:, NCLS:]], axis=1).reshape(B, NCLS)


# speedup vs baseline: 1.0834x; 1.0834x over previous
import functools

import jax
import jax.numpy as jnp
from jax import lax
from jax.experimental import pallas as pl
from jax.experimental.pallas import tpu as pltpu
from jax.experimental.pallas import tpu_sc as plsc

_EPS = 1e-5
_NC, _NS = 2, 16
_NW = _NC * _NS
_LANES = 16


@functools.lru_cache(maxsize=None)
def _make_pool(B, L, D):
    SPW = B // _NW
    NB = 4
    NS = 4
    NCHUNK = SPW // NB
    CROWS = NB * L
    WINDOWS = [(0, 128), (128, 72)]
    NGRP = D // 32
    mesh = plsc.VectorSubcoreMesh(core_axis_name="c", subcore_axis_name="s")

    @functools.partial(
        pl.kernel,
        mesh=mesh,
        compiler_params=pltpu.CompilerParams(
            use_tc_tiling_on_sc=False, needs_layout_passes=False),
        out_type=jax.ShapeDtypeStruct((B * D,), jnp.float32),
        scratch_types=[
            pltpu.VMEM((NS, CROWS), jnp.int32),
            pltpu.VMEM((NS, CROWS, D), jnp.bfloat16),
            pltpu.VMEM((NS, NB * D), jnp.float32),
            pltpu.SemaphoreType.DMA((NS,)),
            pltpu.SemaphoreType.DMA((NS,)),
            pltpu.SemaphoreType.DMA((NS,)),
        ],
    )
    def pool(idx_hbm, emb_hbm, out_hbm, idx_v, rows_v, out_v, isem, gsem, osem):
        wid = lax.axis_index("c") * _NS + lax.axis_index("s")
        base = wid * SPW
        scale = 1.0 / L

        def idx_copy(c, slot):
            pltpu.async_copy(
                idx_hbm.at[pl.ds((base + c * NB) * L, CROWS)],
                idx_v.at[slot], isem.at[slot])

        def idx_wait(slot):
            pltpu.make_async_copy(
                idx_hbm.at[pl.ds(0, CROWS)], idx_v.at[slot], isem.at[slot]
            ).wait()

        def issue_gathers(slot):
            for i in range(NB):
                for off, w in WINDOWS:
                    pltpu.async_copy(
                        emb_hbm.at[idx_v.at[slot, pl.ds(i * L + off, w)]],
                        rows_v.at[slot, pl.ds(i * L + off, w)],
                        gsem.at[slot])

        def drain_gathers(slot):
            pltpu.make_async_copy(
                emb_hbm.at[pl.ds(0, CROWS)], rows_v.at[slot], gsem.at[slot]
            ).wait()

        def store_wait(slot):
            pltpu.make_async_copy(
                out_v.at[slot], out_hbm.at[pl.ds(0, NB * D)], osem.at[slot]
            ).wait()

        for k in range(NS - 1):
            pltpu.sync_copy(
                idx_hbm.at[pl.ds((base + k * NB) * L, CROWS)], idx_v.at[k])
            issue_gathers(k)
        idx_copy(NS - 1, NS - 1)

        @pl.loop(0, NCHUNK, step=NS)
        def _outer(it):
            for p in range(NS):
                c = it + p
                q = (p + NS - 1) % NS

                @pl.when(c + NS - 1 < NCHUNK)
                def _fire_ahead(q=q):
                    idx_wait(q)
                    issue_gathers(q)

                drain_gathers(p)

                @pl.when(c + NS < NCHUNK)
                def _prefetch_idx(c=c, p=p):
                    idx_copy(c + NS, p)

                @pl.when(c >= NS)
                def _drain_prev_store(p=p):
                    store_wait(p)

                for i in range(NB):
                    zero = jnp.zeros((_LANES,), jnp.float32)

                    def rbody(t, accs, i=i, p=p):
                        a = list(accs)
                        r0 = i * L + t * 8
                        for u in range(8):
                            for g in range(NGRP):
                                v = rows_v[p, r0 + u, pl.ds(g * 32, 32)]
                                ev, od = plsc.unpack(
                                    v, format=plsc.PackFormat.INTERLEAVED)
                                a[2 * g] = a[2 * g] + ev
                                a[2 * g + 1] = a[2 * g + 1] + od
                        return tuple(a)

                    accs = lax.fori_loop(0, L // 8, rbody, (zero,) * (2 * NGRP))
                    for k in range(2 * NGRP):
                        out_v[p, pl.ds(i * D + k * _LANES, _LANES)] = (
                            accs[k] * scale)

                pltpu.async_copy(
                    out_v.at[p],
                    out_hbm.at[pl.ds((base + c * NB) * D, NB * D)],
                    osem.at[p])

        for slot in range(NS):
            store_wait(slot)

    return pool


def _mlp(pooled, W1, W2, Wout, bout, alpha, g1, b1, g2, b2):
    B, D = pooled.shape
    F1 = W1.shape[0]
    F2 = W2.shape[0]
    NCLS = Wout.shape[0]
    NP = 32
    bn = (1.0 + _EPS) ** -0.5
    wout_p = jnp.zeros((NP, F2), jnp.float32).at[:NCLS].set(Wout)
    bout_2 = bout.reshape(1, NCLS)
    s1 = (g1 * bn).reshape(1, F1)
    s2 = (g2 * bn).reshape(1, F2)
    b1_2 = b1.reshape(1, F1)
    b2_2 = b2.reshape(1, F2)
    a_2 = alpha.reshape(1, 1)
    bm = 2048

    def body(x_ref, w1_ref, w2_ref, wo_ref, s1_ref, b1_ref, s2_ref, b2_ref,
             bo_ref, a_ref, o_ref):
        x = x_ref[...]
        al = a_ref[0, 0]
        h = lax.dot_general(x, w1_ref[...], (((1,), (1,)), ((), ())),
                            preferred_element_type=jnp.float32)
        h = jnp.where(h > 0, h, al * h)
        h = h * s1_ref[...] + b1_ref[...]
        h = lax.dot_general(h, w2_ref[...], (((1,), (1,)), ((), ())),
                            preferred_element_type=jnp.float32)
        h = jnp.where(h > 0, h, al * h)
        h = h * s2_ref[...] + b2_ref[...]
        o = lax.dot_general(h, wo_ref[...], (((1,), (1,)), ((), ())),
                            preferred_element_type=jnp.float32)
        o_ref[...] = o[:, :o_ref.shape[1]] + bo_ref[...]

    out = pl.pallas_call(
        body,
        grid=(B // bm,),
        in_specs=[
            pl.BlockSpec((bm, D), lambda i: (i, 0)),
            pl.BlockSpec((F1, D), lambda i: (0, 0)),
            pl.BlockSpec((F2, F1), lambda i: (0, 0)),
            pl.BlockSpec((NP, F2), lambda i: (0, 0)),
            pl.BlockSpec((1, F1), lambda i: (0, 0)),
            pl.BlockSpec((1, F1), lambda i: (0, 0)),
            pl.BlockSpec((1, F2), lambda i: (0, 0)),
            pl.BlockSpec((1, F2), lambda i: (0, 0)),
            pl.BlockSpec((1, NCLS), lambda i: (0, 0)),
            pl.BlockSpec((1, 1), lambda i: (0, 0)),
        ],
        out_specs=pl.BlockSpec((bm, NCLS), lambda i: (i, 0)),
        out_shape=jax.ShapeDtypeStruct((B, NCLS), jnp.float32),
    )(pooled, W1, W2, wout_p, s1, b1_2, s2, b2_2, bout_2, a_2)
    return out


def kernel(input, embed, W1, W2, Wout, bout, alpha, g1, b1, g2, b2):
    B, L = input.shape
    V, D = embed.shape
    idx_lin = input.reshape(B * L).astype(jnp.int32)
    emb_lin = embed.astype(jnp.bfloat16).reshape(V * D)
    perm = []
    for g in range(D // 32):
        perm += [g * 32 + 2 * r for r in range(16)]
        perm += [g * 32 + 2 * r + 1 for r in range(16)]
    w1p = W1[:, jnp.array(perm)]
    pooled_lin = _make_pool(B, L, D)(idx_lin, emb_lin.reshape(V, D))
    return _mlp(pooled_lin.reshape(B, D), w1p, W2, Wout, bout, alpha, g1, b1,
                g2, b2)
